# Initial kernel scaffold; baseline (speedup 1.0000x reference)
import functools

import jax
import jax.numpy as jnp
from jax import lax
from jax.experimental import pallas as pl
from jax.experimental.pallas import tpu as pltpu
from jax.experimental.pallas import tpu_sc as plsc

BATCH = 16384
FIELDS = 100
EMBED_DIM = 64
TOTAL = BATCH * FIELDS

NC = 2
NS = 16
NW = NC * NS
PER_W = TOTAL // NW
CHUNK = 128
NCHUNK = PER_W // CHUNK


def _body(x_hbm, w_hbm, out_hbm, idx_v, rows_v, sem):
    wid = lax.axis_index("s") * NC + lax.axis_index("c")
    base = wid * PER_W

    pltpu.sync_copy(x_hbm.at[wid], idx_v)

    def dbl(j, _):
        for i in range(CHUNK // 16):
            v = idx_v[j, pl.ds(i * 16, 16)]
            idx_v[j, pl.ds(i * 16, 16)] = v + v
        return 0

    lax.fori_loop(0, NCHUNK, dbl, 0)

    def chunk(j, _):
        pltpu.async_copy(w_hbm.at[idx_v.at[j]], rows_v, sem).wait()
        pltpu.sync_copy(rows_v, out_hbm.at[pl.ds(base + j * CHUNK, CHUNK)])
        return 0

    lax.fori_loop(0, NCHUNK, chunk, 0)


@jax.jit
def kernel(x, weight):
    x3 = x.reshape(NW, NCHUNK, CHUNK)
    mesh = plsc.VectorSubcoreMesh(core_axis_name="c", subcore_axis_name="s")
    out = pl.kernel(
        _body,
        mesh=mesh,
        out_type=jax.ShapeDtypeStruct((TOTAL, EMBED_DIM), jnp.float32),
        scratch_types=[
            pltpu.VMEM((NCHUNK, CHUNK), jnp.int32),
            pltpu.VMEM((CHUNK, EMBED_DIM), jnp.float32),
            pltpu.SemaphoreType.DMA,
        ],
    )(x3, weight)
    return out.reshape(BATCH, FIELDS, EMBED_DIM)


# speedup vs baseline: 4.9055x; 4.9055x over previous
import functools

import jax
import jax.numpy as jnp
from jax import lax
from jax.experimental import pallas as pl
from jax.experimental.pallas import tpu as pltpu
from jax.experimental.pallas import tpu_sc as plsc

BATCH = 16384
FIELDS = 100
EMBED_DIM = 64
TOTAL = BATCH * FIELDS

NC = 2
NS = 16
NW = NC * NS
PER_W = TOTAL // NW
CHUNK = 128
NCHUNK = PER_W // CHUNK


def _body(x_hbm, w_hbm, out_hbm, idx_v, rows_v, sem):
    wid = lax.axis_index("s") * NC + lax.axis_index("c")
    base = wid * PER_W

    pltpu.sync_copy(x_hbm.at[wid], idx_v)

    def dbl(j, _):
        for i in range(CHUNK // 16):
            v = idx_v[j, pl.ds(i * 16, 16)]
            idx_v[j, pl.ds(i * 16, 16)] = v + v
        return 0

    lax.fori_loop(0, NCHUNK, dbl, 0)

    def chunk(j, _):
        pltpu.async_copy(w_hbm.at[idx_v.at[j]], rows_v, sem).wait()
        pltpu.sync_copy(rows_v, out_hbm.at[pl.ds(base + j * CHUNK, CHUNK)])
        return 0

    lax.fori_loop(0, NCHUNK, chunk, 0)


@jax.jit
def kernel(x, weight):
    x3 = x.reshape(NW, NCHUNK, CHUNK)
    mesh = plsc.VectorSubcoreMesh(core_axis_name="c", subcore_axis_name="s")
    out = pl.kernel(
        _body,
        mesh=mesh,
        out_type=jax.ShapeDtypeStruct((TOTAL, EMBED_DIM), jnp.float32),
        scratch_types=[
            pltpu.VMEM((NCHUNK, CHUNK), jnp.int32),
            pltpu.VMEM((CHUNK, EMBED_DIM), jnp.float32),
            pltpu.SemaphoreType.DMA,
        ],
        compiler_params=pltpu.CompilerParams(use_tc_tiling_on_sc=False),
    )(x3, weight)
    return out.reshape(BATCH, FIELDS, EMBED_DIM)


# speedup vs baseline: 5.6543x; 1.1527x over previous
import functools

import jax
import jax.numpy as jnp
from jax import lax
from jax.experimental import pallas as pl
from jax.experimental.pallas import tpu as pltpu
from jax.experimental.pallas import tpu_sc as plsc

BATCH = 16384
FIELDS = 100
EMBED_DIM = 64
TOTAL = BATCH * FIELDS

NC = 2
NS = 16
NW = NC * NS
PER_W = TOTAL // NW
CHUNK = 128
NCHUNK = PER_W // CHUNK


NBUF = 4
NITER = NCHUNK // NBUF


def _body(x_hbm, w_hbm, out_hbm, idx_v, r0, r1, r2, r3,
          g0, g1, g2, g3, o0, o1, o2, o3):
    rows = (r0, r1, r2, r3)
    gsem = (g0, g1, g2, g3)
    osem = (o0, o1, o2, o3)
    wid = lax.axis_index("s") * NC + lax.axis_index("c")
    base = wid * PER_W

    pltpu.sync_copy(x_hbm.at[wid], idx_v)

    def dbl(j, _):
        for i in range(CHUNK // 16):
            v = idx_v[j, pl.ds(i * 16, 16)]
            idx_v[j, pl.ds(i * 16, 16)] = v + v
        return 0

    lax.fori_loop(0, NCHUNK, dbl, 0)

    for b in range(NBUF):
        pltpu.async_copy(w_hbm.at[idx_v.at[b]], rows[b], gsem[b])

    def rnd(i, _):
        j0 = i * NBUF
        for b in range(NBUF):
            j = j0 + b
            pltpu.make_async_copy(w_hbm.at[idx_v.at[j]], rows[b], gsem[b]).wait()
            pltpu.async_copy(rows[b], out_hbm.at[pl.ds(base + j * CHUNK, CHUNK)],
                             osem[b])
        for b in range(NBUF):
            j = j0 + b
            pltpu.make_async_copy(
                rows[b], out_hbm.at[pl.ds(base + j * CHUNK, CHUNK)], osem[b]
            ).wait()

            @pl.when(i + 1 < NITER)
            def _():
                pltpu.async_copy(w_hbm.at[idx_v.at[j + NBUF]], rows[b], gsem[b])

        return 0

    lax.fori_loop(0, NITER, rnd, 0)


@jax.jit
def kernel(x, weight):
    x3 = x.reshape(NW, NCHUNK, CHUNK)
    mesh = plsc.VectorSubcoreMesh(core_axis_name="c", subcore_axis_name="s")
    out = pl.kernel(
        _body,
        mesh=mesh,
        out_type=jax.ShapeDtypeStruct((TOTAL, EMBED_DIM), jnp.float32),
        scratch_types=[
            pltpu.VMEM((NCHUNK, CHUNK), jnp.int32),
        ] + [pltpu.VMEM((CHUNK, EMBED_DIM), jnp.float32) for _ in range(NBUF)]
        + [pltpu.SemaphoreType.DMA for _ in range(2 * NBUF)],
        compiler_params=pltpu.CompilerParams(use_tc_tiling_on_sc=False),
    )(x3, weight)
    return out.reshape(BATCH, FIELDS, EMBED_DIM)
